# Initial kernel scaffold; baseline (speedup 1.0000x reference)
import jax
import jax.numpy as jnp
from jax.experimental import pallas as pl


def kernel(x, xgboost_score, pos, batch, params):
    raise NotImplementedError("write your pallas kernel here")


# speedup vs baseline: 1.6310x; 1.6310x over previous
import functools

import jax
import jax.numpy as jnp
from jax.experimental import pallas as pl
from jax.experimental.pallas import tpu as pltpu

_CUT = 0.00394
_BN_EPS = 1e-5
_N = 10000


def _rup(x, m):
    return (x + m - 1) // m * m


def _fps_body(p_ref, cnt_ref, sel_ref, *, R, S, n_out):
    cnt = cnt_ref[0, 0]
    px = p_ref[0:R, :]
    py = p_ref[R:2 * R, :]
    pz = p_ref[2 * R:3 * R, :]
    idx = (jax.lax.broadcasted_iota(jnp.int32, (R, 128), 0) * 128
           + jax.lax.broadcasted_iota(jnp.int32, (R, 128), 1))
    sel_idx = (jax.lax.broadcasted_iota(jnp.int32, (S, 128), 0) * 128
               + jax.lax.broadcasted_iota(jnp.int32, (S, 128), 1))
    inf = jnp.float32(jnp.inf)
    dist0 = jnp.where(idx < cnt, inf, -inf)
    sel0 = jnp.zeros((S, 128), jnp.int32)

    def body(i, carry):
        dist, sel, cur = carry
        sel = jnp.where(sel_idx == i, cur, sel)
        curmask = idx == cur
        cx = jnp.sum(jnp.where(curmask, px, 0.0))
        cy = jnp.sum(jnp.where(curmask, py, 0.0))
        cz = jnp.sum(jnp.where(curmask, pz, 0.0))
        dx = px - cx
        dy = py - cy
        dz = pz - cz
        d = dx * dx + dy * dy + dz * dz
        dist = jnp.minimum(dist, d)
        m = jnp.max(dist)
        cur = jnp.min(jnp.where(dist == m, idx, jnp.int32(R * 128)))
        return dist, sel, cur

    _, sel, _ = jax.lax.fori_loop(0, n_out, body, (dist0, sel0, jnp.int32(0)),
                                  unroll=False)
    sel_ref[...] = sel


def _fps(p, cnt, n_out):
    n = p.shape[0]
    n_pad = _rup(n, 128)
    R = n_pad // 128
    S = _rup(pl.cdiv(n_out, 128), 8)
    pt = jnp.zeros((3, n_pad), jnp.float32).at[:, :n].set(p.T)
    pt = pt.reshape(3 * R, 128)
    cnt2 = jnp.asarray(cnt, jnp.int32).reshape(1, 1)
    out = pl.pallas_call(
        functools.partial(_fps_body, R=R, S=S, n_out=n_out),
        out_shape=jax.ShapeDtypeStruct((S, 128), jnp.int32),
        in_specs=[
            pl.BlockSpec(memory_space=pltpu.VMEM),
            pl.BlockSpec(memory_space=pltpu.SMEM),
        ],
        out_specs=pl.BlockSpec(memory_space=pltpu.VMEM),
    )(pt, cnt2)
    return out.reshape(-1)[:n_out]


def _conv_body(msg_ref, ev_ref, *args, T, K, nlayers):
    wr = args[:-1]
    out_ref = args[-1]
    h = msg_ref[...]
    for li in range(nlayers):
        w = wr[4 * li][...]
        b = wr[4 * li + 1][...]
        g = wr[4 * li + 2][...]
        be = wr[4 * li + 3][...]
        h = jnp.dot(h, w, preferred_element_type=jnp.float32) + b
        h = jnp.maximum(h, 0.0)
        h = (h / jnp.sqrt(jnp.float32(1.0 + _BN_EPS))) * g + be
    ev = ev_ref[...]
    h = jnp.where(ev > 0, h, -jnp.inf)
    for t in range(T):
        m = jnp.max(h[t * K:(t + 1) * K, :], axis=0, keepdims=True)
        out_ref[t:t + 1, :] = jnp.where(m == -jnp.inf, 0.0, m)


def _sa_conv(ps, x_src, pos_src, pos_dst, cols, ev, T=16, K=72):
    nd, k0 = cols.shape
    ndp = _rup(nd, T)
    cols_p = jnp.zeros((ndp, K), jnp.int32).at[:nd, :k0].set(cols)
    ev_p = jnp.zeros((ndp, K), jnp.bool_).at[:nd, :k0].set(ev)
    pos_dst_p = jnp.zeros((ndp, 3), pos_dst.dtype).at[:nd].set(pos_dst)
    col = cols_p.reshape(-1)
    row = jnp.repeat(jnp.arange(ndp, dtype=jnp.int32), K)
    msg = jnp.concatenate([x_src[col], pos_src[col] - pos_dst_p[row]], axis=1)
    evm = ev_p.reshape(-1, 1).astype(jnp.float32)
    cin = msg.shape[1]
    cout = ps[-1][0].shape[1]
    wr = []
    wspecs = []
    for (w, b, g, be) in ps:
        for a in (w, b.reshape(1, -1), g.reshape(1, -1), be.reshape(1, -1)):
            wr.append(a)
            wspecs.append(pl.BlockSpec(a.shape, lambda i: (0, 0)))
    grid = ndp // T
    out = pl.pallas_call(
        functools.partial(_conv_body, T=T, K=K, nlayers=len(ps)),
        grid=(grid,),
        out_shape=jax.ShapeDtypeStruct((ndp, cout), jnp.float32),
        in_specs=[
            pl.BlockSpec((T * K, cin), lambda i: (i, 0)),
            pl.BlockSpec((T * K, 1), lambda i: (i, 0)),
        ] + wspecs,
        out_specs=pl.BlockSpec((T, cout), lambda i: (i, 0)),
    )(msg, evm, *wr)
    return out[:nd]


def _mlp_body(x_ref, *args, specs):
    wr = args[:-1]
    out_ref = args[-1]
    h = x_ref[...]
    wi = 0
    for kind in specs:
        w = wr[wi][...]
        b = wr[wi + 1][...]
        wi += 2
        h = jnp.dot(h, w, preferred_element_type=jnp.float32) + b
        if kind == 'bn':
            h = jnp.maximum(h, 0.0)
            g = wr[wi][...]
            be = wr[wi + 1][...]
            wi += 2
            h = (h / jnp.sqrt(jnp.float32(1.0 + _BN_EPS))) * g + be
        elif kind == 'relu':
            h = jnp.maximum(h, 0.0)
    out_ref[...] = h


def _mlp(x, layers, tr=256):
    rn, cin = x.shape
    rp = _rup(rn, tr)
    xp = jnp.zeros((rp, cin), jnp.float32).at[:rn].set(x)
    wr = []
    wspecs = []
    specs = []
    for kind, ws in layers:
        specs.append(kind)
        tensors = [ws[0], ws[1].reshape(1, -1)]
        if kind == 'bn':
            tensors += [ws[2].reshape(1, -1), ws[3].reshape(1, -1)]
        for a in tensors:
            wr.append(a)
            wspecs.append(pl.BlockSpec(a.shape, lambda i: (0, 0)))
    cout = layers[-1][1][0].shape[1]
    grid = rp // tr
    out = pl.pallas_call(
        functools.partial(_mlp_body, specs=tuple(specs)),
        grid=(grid,),
        out_shape=jax.ShapeDtypeStruct((rp, cout), jnp.float32),
        in_specs=[pl.BlockSpec((tr, cin), lambda i: (i, 0))] + wspecs,
        out_specs=pl.BlockSpec((tr, cout), lambda i: (i, 0)),
    )(xp, *wr)
    return out[:rn]


def _radius(src_pos, cnt_src, dst_pos, cnt_dst, r2, max_nn=64):
    ns = src_pos.shape[0]
    nd = dst_pos.shape[0]
    d2 = ((dst_pos[:, None, :] - src_pos[None, :, :]) ** 2).sum(-1)
    j = jnp.arange(ns, dtype=jnp.int32)
    hit = (d2 <= r2) & (j[None, :] < cnt_src)
    key = jnp.where(hit, j[None, :], jnp.int32(ns))
    neg, _ = jax.lax.top_k(-key, max_nn)
    cols = -neg
    i = jnp.arange(nd, dtype=jnp.int32)[:, None]
    ev = (cols < ns) & (cols != i) & (i < cnt_dst)
    cols = jnp.concatenate(
        [jnp.minimum(cols, ns - 1), jnp.broadcast_to(i, (nd, 1))], axis=1)
    ev = jnp.concatenate([ev, i < cnt_dst], axis=1)
    return cols, ev


def _knn(src_pos, cnt_src, dst_pos, k):
    d2 = ((dst_pos[:, None, :] - src_pos[None, :, :]) ** 2).sum(-1)
    j = jnp.arange(src_pos.shape[0])
    d2 = jnp.where(j[None, :] < cnt_src, d2, jnp.inf)
    _, idx = jax.lax.top_k(-d2, k)
    return idx


def _interp(x_src, pos_src, pos_dst, nn_idx):
    diff = pos_src[nn_idx] - pos_dst[:, None, :]
    d2 = (diff * diff).sum(-1, keepdims=True)
    w = 1.0 / jnp.maximum(d2, 1e-16)
    return (x_src[nn_idx] * w).sum(1) / w.sum(1)


def kernel(x, xgboost_score, pos, batch, params):
    score = xgboost_score
    over = score > _CUT
    mask_idx = jnp.nonzero(over, size=_N, fill_value=_N)[0].astype(jnp.int32)
    cnt0 = jnp.sum(over).astype(jnp.int32)
    p0 = pos[mask_idx]
    n1s = (_N + 1) // 2
    n2s = (2 * n1s + 4) // 5
    n3s = (2 * n2s + 4) // 5

    sel1 = _fps(p0, cnt0, n1s)
    n1 = jnp.maximum(1, (cnt0 + 1) // 2)
    p1 = p0[sel1]
    c1, v1 = _radius(p0, cnt0, p1, n1, 49.0)
    sel2 = _fps(p1, n1, n2s)
    n2 = jnp.maximum(1, (2 * n1 + 4) // 5)
    p2 = p1[sel2]
    c2, v2 = _radius(p1, n1, p2, n2, 256.0)
    sel3 = _fps(p2, n2, n3s)
    n3 = jnp.maximum(1, (2 * n2 + 4) // 5)
    p3 = p2[sel3]
    c3, v3 = _radius(p2, n2, p3, n3, 900.0)
    nn3 = jnp.zeros((n3s, 1), jnp.int32)
    nn2 = _knn(p3, n3, p2, 3)
    nn1 = _knn(p2, n2, p1, 3)
    nn0 = _knn(p1, n1, p0, 3)

    x0 = jnp.concatenate([x[mask_idx], score[mask_idx][:, None]], axis=1)
    x1 = _sa_conv(params['sa1'], x0, p0, p1, c1, v1)
    x2 = _sa_conv(params['sa2'], x1, p1, p2, c2, v2)
    x3 = _sa_conv(params['sa3'], x2, p2, p3, c3, v3)
    h4 = _mlp(jnp.concatenate([x3, p3], axis=1),
              [('bn', w) for w in params['sa4']])
    valid3 = jnp.arange(h4.shape[0]) < n3
    x4 = jnp.max(jnp.where(valid3[:, None], h4, -jnp.inf), axis=0,
                 keepdims=True)
    p4 = jnp.zeros((1, 3), x.dtype)
    y = _interp(x4, p4, p3, nn3)
    y = _mlp(jnp.concatenate([y, x3], axis=1),
             [('bn', w) for w in params['fp3']])
    y = _interp(y, p3, p2, nn2)
    y = _mlp(jnp.concatenate([y, x2], axis=1),
             [('bn', w) for w in params['fp2']])
    y = _interp(y, p2, p1, nn1)
    y = _mlp(jnp.concatenate([y, x1], axis=1),
             [('bn', w) for w in params['fp1']])
    y = _interp(y, p1, p0, nn0)
    head = ([('bn', w) for w in params['fp0']]
            + [('relu', params['lin1']), ('none', params['lin2']),
               ('none', params['lin3'])])
    h = _mlp(jnp.concatenate([y, x0], axis=1), head)
    out = jnp.zeros((pos.shape[0],), x.dtype).at[mask_idx].set(
        h[:, 0], mode='drop')
    return jax.nn.sigmoid(out)


# speedup vs baseline: 7.8950x; 4.8406x over previous
import functools

import jax
import jax.numpy as jnp
from jax.experimental import pallas as pl
from jax.experimental.pallas import tpu as pltpu

_CUT = 0.00394
_BN_EPS = 1e-5
_N = 10000


def _rup(x, m):
    return (x + m - 1) // m * m


def _fps_body(p_ref, cnt_ref, sel_ref, *, R, S, n_out):
    cnt = cnt_ref[0, 0]
    px = p_ref[0:R, :]
    py = p_ref[R:2 * R, :]
    pz = p_ref[2 * R:3 * R, :]
    idx = (jax.lax.broadcasted_iota(jnp.int32, (R, 128), 0) * 128
           + jax.lax.broadcasted_iota(jnp.int32, (R, 128), 1))
    sel_idx = (jax.lax.broadcasted_iota(jnp.int32, (S, 128), 0) * 128
               + jax.lax.broadcasted_iota(jnp.int32, (S, 128), 1))
    inf = jnp.float32(jnp.inf)
    dist0 = jnp.where(idx < cnt, inf, -inf)
    sel0 = jnp.zeros((S, 128), jnp.int32)

    def body(i, carry):
        dist, sel, cur = carry
        sel = jnp.where(sel_idx == i, cur, sel)
        curmask = idx == cur
        cx = jnp.sum(jnp.where(curmask, px, 0.0))
        cy = jnp.sum(jnp.where(curmask, py, 0.0))
        cz = jnp.sum(jnp.where(curmask, pz, 0.0))
        dx = px - cx
        dy = py - cy
        dz = pz - cz
        d = dx * dx + dy * dy + dz * dz
        dist = jnp.minimum(dist, d)
        m = jnp.max(dist)
        cur = jnp.min(jnp.where(dist == m, idx, jnp.int32(R * 128)))
        return dist, sel, cur

    _, sel, _ = jax.lax.fori_loop(0, n_out, body, (dist0, sel0, jnp.int32(0)),
                                  unroll=False)
    sel_ref[...] = sel


def _fps(p, cnt, n_out):
    n = p.shape[0]
    n_pad = _rup(n, 128)
    R = n_pad // 128
    S = _rup(pl.cdiv(n_out, 128), 8)
    pt = jnp.zeros((3, n_pad), jnp.float32).at[:, :n].set(p.T)
    pt = pt.reshape(3 * R, 128)
    cnt2 = jnp.asarray(cnt, jnp.int32).reshape(1, 1)
    out = pl.pallas_call(
        functools.partial(_fps_body, R=R, S=S, n_out=n_out),
        out_shape=jax.ShapeDtypeStruct((S, 128), jnp.int32),
        in_specs=[
            pl.BlockSpec(memory_space=pltpu.VMEM),
            pl.BlockSpec(memory_space=pltpu.SMEM),
        ],
        out_specs=pl.BlockSpec(memory_space=pltpu.VMEM),
    )(pt, cnt2)
    return out.reshape(-1)[:n_out]


def _conv_body(msg_ref, ev_ref, *args, T, K, nlayers):
    wr = args[:-1]
    out_ref = args[-1]
    h = msg_ref[...]
    for li in range(nlayers):
        w = wr[4 * li][...]
        b = wr[4 * li + 1][...]
        g = wr[4 * li + 2][...]
        be = wr[4 * li + 3][...]
        h = jnp.dot(h, w, preferred_element_type=jnp.float32) + b
        h = jnp.maximum(h, 0.0)
        h = (h / jnp.sqrt(jnp.float32(1.0 + _BN_EPS))) * g + be
    ev = ev_ref[...]
    h = jnp.where(ev > 0, h, -jnp.inf)
    for t in range(T):
        m = jnp.max(h[t * K:(t + 1) * K, :], axis=0, keepdims=True)
        out_ref[t:t + 1, :] = jnp.where(m == -jnp.inf, 0.0, m)


def _sa_conv(ps, x_src, pos_src, pos_dst, cols, ev, T=16, K=72):
    nd, k0 = cols.shape
    ndp = _rup(nd, T)
    cols_p = jnp.zeros((ndp, K), jnp.int32).at[:nd, :k0].set(cols)
    ev_p = jnp.zeros((ndp, K), jnp.bool_).at[:nd, :k0].set(ev)
    pos_dst_p = jnp.zeros((ndp, 3), pos_dst.dtype).at[:nd].set(pos_dst)
    col = cols_p.reshape(-1)
    row = jnp.repeat(jnp.arange(ndp, dtype=jnp.int32), K)
    msg = jnp.concatenate([x_src[col], pos_src[col] - pos_dst_p[row]], axis=1)
    evm = ev_p.reshape(-1, 1).astype(jnp.float32)
    cin = msg.shape[1]
    cout = ps[-1][0].shape[1]
    wr = []
    wspecs = []
    for (w, b, g, be) in ps:
        for a in (w, b.reshape(1, -1), g.reshape(1, -1), be.reshape(1, -1)):
            wr.append(a)
            wspecs.append(pl.BlockSpec(a.shape, lambda i: (0, 0)))
    grid = ndp // T
    out = pl.pallas_call(
        functools.partial(_conv_body, T=T, K=K, nlayers=len(ps)),
        grid=(grid,),
        out_shape=jax.ShapeDtypeStruct((ndp, cout), jnp.float32),
        in_specs=[
            pl.BlockSpec((T * K, cin), lambda i: (i, 0)),
            pl.BlockSpec((T * K, 1), lambda i: (i, 0)),
        ] + wspecs,
        out_specs=pl.BlockSpec((T, cout), lambda i: (i, 0)),
    )(msg, evm, *wr)
    return out[:nd]


def _mlp_body(x_ref, *args, specs):
    wr = args[:-1]
    out_ref = args[-1]
    h = x_ref[...]
    wi = 0
    for kind in specs:
        w = wr[wi][...]
        b = wr[wi + 1][...]
        wi += 2
        h = jnp.dot(h, w, preferred_element_type=jnp.float32) + b
        if kind == 'bn':
            h = jnp.maximum(h, 0.0)
            g = wr[wi][...]
            be = wr[wi + 1][...]
            wi += 2
            h = (h / jnp.sqrt(jnp.float32(1.0 + _BN_EPS))) * g + be
        elif kind == 'relu':
            h = jnp.maximum(h, 0.0)
    out_ref[...] = h


def _mlp(x, layers, tr=256):
    rn, cin = x.shape
    rp = _rup(rn, tr)
    xp = jnp.zeros((rp, cin), jnp.float32).at[:rn].set(x)
    wr = []
    wspecs = []
    specs = []
    for kind, ws in layers:
        specs.append(kind)
        tensors = [ws[0], ws[1].reshape(1, -1)]
        if kind == 'bn':
            tensors += [ws[2].reshape(1, -1), ws[3].reshape(1, -1)]
        for a in tensors:
            wr.append(a)
            wspecs.append(pl.BlockSpec(a.shape, lambda i: (0, 0)))
    cout = layers[-1][1][0].shape[1]
    grid = rp // tr
    out = pl.pallas_call(
        functools.partial(_mlp_body, specs=tuple(specs)),
        grid=(grid,),
        out_shape=jax.ShapeDtypeStruct((rp, cout), jnp.float32),
        in_specs=[pl.BlockSpec((tr, cin), lambda i: (i, 0))] + wspecs,
        out_specs=pl.BlockSpec((tr, cout), lambda i: (i, 0)),
    )(xp, *wr)
    return out[:rn]


def _hitcum_body(dst_ref, src_ref, cnt_ref, out_ref, *, r2):
    cnt = cnt_ref[0, 0]
    sx = src_ref[0:1, :]
    sy = src_ref[1:2, :]
    sz = src_ref[2:3, :]
    dx = dst_ref[:, 0:1] - sx
    dy = dst_ref[:, 1:2] - sy
    dz = dst_ref[:, 2:3] - sz
    d2 = dx * dx + dy * dy + dz * dz
    j = jax.lax.broadcasted_iota(jnp.int32, d2.shape, 1)
    hit = (d2 <= r2) & (j < cnt)
    c = hit.astype(jnp.int32)
    n = c.shape[1]
    s = 1
    while s < n:
        shifted = jnp.concatenate(
            [jnp.zeros((c.shape[0], s), jnp.int32), c[:, :-s]], axis=1)
        c = c + shifted
        s *= 2
    out_ref[...] = c


def _radius(src_pos, cnt_src, dst_pos, cnt_dst, r2, max_nn=64):
    ns = src_pos.shape[0]
    nd = dst_pos.shape[0]
    tr = 32
    nsp = _rup(ns, 128)
    ndp = _rup(nd, tr)
    srcp = jnp.zeros((3, nsp), jnp.float32).at[:, :ns].set(src_pos.T)
    dstp = jnp.zeros((ndp, 3), jnp.float32).at[:nd].set(dst_pos)
    cnt2 = jnp.asarray(cnt_src, jnp.int32).reshape(1, 1)
    c = pl.pallas_call(
        functools.partial(_hitcum_body, r2=r2),
        grid=(ndp // tr,),
        out_shape=jax.ShapeDtypeStruct((ndp, nsp), jnp.int32),
        in_specs=[
            pl.BlockSpec((tr, 3), lambda i: (i, 0)),
            pl.BlockSpec((3, nsp), lambda i: (0, 0)),
            pl.BlockSpec(memory_space=pltpu.SMEM),
        ],
        out_specs=pl.BlockSpec((tr, nsp), lambda i: (i, 0)),
    )(dstp, srcp, cnt2)
    c = c[:nd, :ns]
    kq = jnp.broadcast_to(jnp.arange(1, max_nn + 1, dtype=jnp.int32)[None, :],
                          (nd, max_nn))
    cols = jax.vmap(jnp.searchsorted)(c, kq).astype(jnp.int32)
    i = jnp.arange(nd, dtype=jnp.int32)[:, None]
    ev = (cols < ns) & (cols != i) & (i < cnt_dst)
    cols = jnp.concatenate(
        [jnp.minimum(cols, ns - 1), jnp.broadcast_to(i, (nd, 1))], axis=1)
    ev = jnp.concatenate([ev, i < cnt_dst], axis=1)
    return cols, ev


def _knn(src_pos, cnt_src, dst_pos, k):
    d2 = ((dst_pos[:, None, :] - src_pos[None, :, :]) ** 2).sum(-1)
    j = jnp.arange(src_pos.shape[0])
    d2 = jnp.where(j[None, :] < cnt_src, d2, jnp.inf)
    _, idx = jax.lax.top_k(-d2, k)
    return idx


def _interp(x_src, pos_src, pos_dst, nn_idx):
    diff = pos_src[nn_idx] - pos_dst[:, None, :]
    d2 = (diff * diff).sum(-1, keepdims=True)
    w = 1.0 / jnp.maximum(d2, 1e-16)
    return (x_src[nn_idx] * w).sum(1) / w.sum(1)


def kernel(x, xgboost_score, pos, batch, params):
    score = xgboost_score
    over = score > _CUT
    mask_idx = jnp.nonzero(over, size=_N, fill_value=_N)[0].astype(jnp.int32)
    cnt0 = jnp.sum(over).astype(jnp.int32)
    p0 = pos[mask_idx]
    n1s = (_N + 1) // 2
    n2s = (2 * n1s + 4) // 5
    n3s = (2 * n2s + 4) // 5

    sel1 = _fps(p0, cnt0, n1s)
    n1 = jnp.maximum(1, (cnt0 + 1) // 2)
    p1 = p0[sel1]
    c1, v1 = _radius(p0, cnt0, p1, n1, 49.0)
    sel2 = _fps(p1, n1, n2s)
    n2 = jnp.maximum(1, (2 * n1 + 4) // 5)
    p2 = p1[sel2]
    c2, v2 = _radius(p1, n1, p2, n2, 256.0)
    sel3 = _fps(p2, n2, n3s)
    n3 = jnp.maximum(1, (2 * n2 + 4) // 5)
    p3 = p2[sel3]
    c3, v3 = _radius(p2, n2, p3, n3, 900.0)
    nn3 = jnp.zeros((n3s, 1), jnp.int32)
    nn2 = _knn(p3, n3, p2, 3)
    nn1 = _knn(p2, n2, p1, 3)
    nn0 = _knn(p1, n1, p0, 3)

    x0 = jnp.concatenate([x[mask_idx], score[mask_idx][:, None]], axis=1)
    x1 = _sa_conv(params['sa1'], x0, p0, p1, c1, v1)
    x2 = _sa_conv(params['sa2'], x1, p1, p2, c2, v2)
    x3 = _sa_conv(params['sa3'], x2, p2, p3, c3, v3)
    h4 = _mlp(jnp.concatenate([x3, p3], axis=1),
              [('bn', w) for w in params['sa4']])
    valid3 = jnp.arange(h4.shape[0]) < n3
    x4 = jnp.max(jnp.where(valid3[:, None], h4, -jnp.inf), axis=0,
                 keepdims=True)
    p4 = jnp.zeros((1, 3), x.dtype)
    y = _interp(x4, p4, p3, nn3)
    y = _mlp(jnp.concatenate([y, x3], axis=1),
             [('bn', w) for w in params['fp3']])
    y = _interp(y, p3, p2, nn2)
    y = _mlp(jnp.concatenate([y, x2], axis=1),
             [('bn', w) for w in params['fp2']])
    y = _interp(y, p2, p1, nn1)
    y = _mlp(jnp.concatenate([y, x1], axis=1),
             [('bn', w) for w in params['fp1']])
    y = _interp(y, p1, p0, nn0)
    head = ([('bn', w) for w in params['fp0']]
            + [('relu', params['lin1']), ('none', params['lin2']),
               ('none', params['lin3'])])
    h = _mlp(jnp.concatenate([y, x0], axis=1), head)
    out = jnp.zeros((pos.shape[0],), x.dtype).at[mask_idx].set(
        h[:, 0], mode='drop')
    return jax.nn.sigmoid(out)


# speedup vs baseline: 7.9459x; 1.0064x over previous
import functools

import jax
import jax.numpy as jnp
from jax.experimental import pallas as pl
from jax.experimental.pallas import tpu as pltpu

_CUT = 0.00394
_BN_EPS = 1e-5
_N = 10000


def _rup(x, m):
    return (x + m - 1) // m * m


def _fps_body(p_ref, cnt_ref, sel_ref, *, R, S, n_out):
    cnt = cnt_ref[0, 0]
    px = p_ref[0:R, :]
    py = p_ref[R:2 * R, :]
    pz = p_ref[2 * R:3 * R, :]
    idx = (jax.lax.broadcasted_iota(jnp.int32, (R, 128), 0) * 128
           + jax.lax.broadcasted_iota(jnp.int32, (R, 128), 1))
    sel_idx = (jax.lax.broadcasted_iota(jnp.int32, (S, 128), 0) * 128
               + jax.lax.broadcasted_iota(jnp.int32, (S, 128), 1))
    inf = jnp.float32(jnp.inf)
    dist0 = jnp.where(idx < cnt, inf, -inf)
    sel0 = jnp.zeros((S, 128), jnp.int32)

    def body(i, carry):
        dist, sel, cur = carry
        sel = jnp.where(sel_idx == i, cur, sel)
        curmask = idx == cur
        cx = jnp.sum(jnp.where(curmask, px, 0.0))
        cy = jnp.sum(jnp.where(curmask, py, 0.0))
        cz = jnp.sum(jnp.where(curmask, pz, 0.0))
        dx = px - cx
        dy = py - cy
        dz = pz - cz
        d = dx * dx + dy * dy + dz * dz
        dist = jnp.minimum(dist, d)
        m = jnp.max(dist)
        cur = jnp.min(jnp.where(dist == m, idx, jnp.int32(R * 128)))
        return dist, sel, cur

    _, sel, _ = jax.lax.fori_loop(0, n_out, body, (dist0, sel0, jnp.int32(0)),
                                  unroll=False)
    sel_ref[...] = sel


def _fps(p, cnt, n_out):
    n = p.shape[0]
    n_pad = _rup(n, 128)
    R = n_pad // 128
    S = _rup(pl.cdiv(n_out, 128), 8)
    pt = jnp.zeros((3, n_pad), jnp.float32).at[:, :n].set(p.T)
    pt = pt.reshape(3 * R, 128)
    cnt2 = jnp.asarray(cnt, jnp.int32).reshape(1, 1)
    out = pl.pallas_call(
        functools.partial(_fps_body, R=R, S=S, n_out=n_out),
        out_shape=jax.ShapeDtypeStruct((S, 128), jnp.int32),
        in_specs=[
            pl.BlockSpec(memory_space=pltpu.VMEM),
            pl.BlockSpec(memory_space=pltpu.SMEM),
        ],
        out_specs=pl.BlockSpec(memory_space=pltpu.VMEM),
    )(pt, cnt2)
    return out.reshape(-1)[:n_out]


def _conv_body(msg_ref, ev_ref, *args, T, K, nlayers):
    wr = args[:-1]
    out_ref = args[-1]
    h = msg_ref[...]
    for li in range(nlayers):
        w = wr[4 * li][...]
        b = wr[4 * li + 1][...]
        g = wr[4 * li + 2][...]
        be = wr[4 * li + 3][...]
        h = jnp.dot(h, w, preferred_element_type=jnp.float32) + b
        h = jnp.maximum(h, 0.0)
        h = (h / jnp.sqrt(jnp.float32(1.0 + _BN_EPS))) * g + be
    ev = ev_ref[...]
    h = jnp.where(ev > 0, h, -jnp.inf)
    for t in range(T):
        m = jnp.max(h[t * K:(t + 1) * K, :], axis=0, keepdims=True)
        out_ref[t:t + 1, :] = jnp.where(m == -jnp.inf, 0.0, m)


def _sa_conv(ps, x_src, pos_src, pos_dst, cols, ev, T=32, K=72):
    nd, k0 = cols.shape
    ndp = _rup(nd, T)
    cols_p = jnp.zeros((ndp, K), jnp.int32).at[:nd, :k0].set(cols)
    ev_p = jnp.zeros((ndp, K), jnp.bool_).at[:nd, :k0].set(ev)
    pos_dst_p = jnp.zeros((ndp, 3), pos_dst.dtype).at[:nd].set(pos_dst)
    col = cols_p.reshape(-1)
    row = jnp.repeat(jnp.arange(ndp, dtype=jnp.int32), K)
    msg = jnp.concatenate([x_src[col], pos_src[col] - pos_dst_p[row]], axis=1)
    evm = ev_p.reshape(-1, 1).astype(jnp.float32)
    cin = msg.shape[1]
    cout = ps[-1][0].shape[1]
    wr = []
    wspecs = []
    for (w, b, g, be) in ps:
        for a in (w, b.reshape(1, -1), g.reshape(1, -1), be.reshape(1, -1)):
            wr.append(a)
            wspecs.append(pl.BlockSpec(a.shape, lambda i: (0, 0)))
    grid = ndp // T
    out = pl.pallas_call(
        functools.partial(_conv_body, T=T, K=K, nlayers=len(ps)),
        grid=(grid,),
        out_shape=jax.ShapeDtypeStruct((ndp, cout), jnp.float32),
        in_specs=[
            pl.BlockSpec((T * K, cin), lambda i: (i, 0)),
            pl.BlockSpec((T * K, 1), lambda i: (i, 0)),
        ] + wspecs,
        out_specs=pl.BlockSpec((T, cout), lambda i: (i, 0)),
    )(msg, evm, *wr)
    return out[:nd]


def _mlp_body(x_ref, *args, specs):
    wr = args[:-1]
    out_ref = args[-1]
    h = x_ref[...]
    wi = 0
    for kind in specs:
        w = wr[wi][...]
        b = wr[wi + 1][...]
        wi += 2
        h = jnp.dot(h, w, preferred_element_type=jnp.float32) + b
        if kind == 'bn':
            h = jnp.maximum(h, 0.0)
            g = wr[wi][...]
            be = wr[wi + 1][...]
            wi += 2
            h = (h / jnp.sqrt(jnp.float32(1.0 + _BN_EPS))) * g + be
        elif kind == 'relu':
            h = jnp.maximum(h, 0.0)
    out_ref[...] = h


def _mlp(x, layers, tr=512):
    rn, cin = x.shape
    rp = _rup(rn, tr)
    xp = jnp.zeros((rp, cin), jnp.float32).at[:rn].set(x)
    wr = []
    wspecs = []
    specs = []
    for kind, ws in layers:
        specs.append(kind)
        tensors = [ws[0], ws[1].reshape(1, -1)]
        if kind == 'bn':
            tensors += [ws[2].reshape(1, -1), ws[3].reshape(1, -1)]
        for a in tensors:
            wr.append(a)
            wspecs.append(pl.BlockSpec(a.shape, lambda i: (0, 0)))
    cout = layers[-1][1][0].shape[1]
    grid = rp // tr
    out = pl.pallas_call(
        functools.partial(_mlp_body, specs=tuple(specs)),
        grid=(grid,),
        out_shape=jax.ShapeDtypeStruct((rp, cout), jnp.float32),
        in_specs=[pl.BlockSpec((tr, cin), lambda i: (i, 0))] + wspecs,
        out_specs=pl.BlockSpec((tr, cout), lambda i: (i, 0)),
    )(xp, *wr)
    return out[:rn]


def _hitcum_body(dst_ref, src_ref, cnt_ref, out_ref, *, r2):
    cnt = cnt_ref[0, 0]
    sx = src_ref[0:1, :]
    sy = src_ref[1:2, :]
    sz = src_ref[2:3, :]
    dx = dst_ref[:, 0:1] - sx
    dy = dst_ref[:, 1:2] - sy
    dz = dst_ref[:, 2:3] - sz
    d2 = dx * dx + dy * dy + dz * dz
    j = jax.lax.broadcasted_iota(jnp.int32, d2.shape, 1)
    hit = (d2 <= r2) & (j < cnt)
    c = hit.astype(jnp.int32)
    n = c.shape[1]
    s = 1
    while s < n:
        shifted = jnp.concatenate(
            [jnp.zeros((c.shape[0], s), jnp.int32), c[:, :-s]], axis=1)
        c = c + shifted
        s *= 2
    out_ref[...] = c.astype(jnp.int16)


def _radius(src_pos, cnt_src, dst_pos, cnt_dst, r2, max_nn=64):
    ns = src_pos.shape[0]
    nd = dst_pos.shape[0]
    tr = 32
    nsp = _rup(ns, 128)
    ndp = _rup(nd, tr)
    srcp = jnp.zeros((3, nsp), jnp.float32).at[:, :ns].set(src_pos.T)
    dstp = jnp.zeros((ndp, 3), jnp.float32).at[:nd].set(dst_pos)
    cnt2 = jnp.asarray(cnt_src, jnp.int32).reshape(1, 1)
    c = pl.pallas_call(
        functools.partial(_hitcum_body, r2=r2),
        grid=(ndp // tr,),
        out_shape=jax.ShapeDtypeStruct((ndp, nsp), jnp.int16),
        in_specs=[
            pl.BlockSpec((tr, 3), lambda i: (i, 0)),
            pl.BlockSpec((3, nsp), lambda i: (0, 0)),
            pl.BlockSpec(memory_space=pltpu.SMEM),
        ],
        out_specs=pl.BlockSpec((tr, nsp), lambda i: (i, 0)),
    )(dstp, srcp, cnt2)
    c = c[:nd, :ns]
    kq = jnp.broadcast_to(jnp.arange(1, max_nn + 1, dtype=jnp.int16)[None, :],
                          (nd, max_nn))
    cols = jax.vmap(jnp.searchsorted)(c, kq).astype(jnp.int32)
    i = jnp.arange(nd, dtype=jnp.int32)[:, None]
    ev = (cols < ns) & (cols != i) & (i < cnt_dst)
    cols = jnp.concatenate(
        [jnp.minimum(cols, ns - 1), jnp.broadcast_to(i, (nd, 1))], axis=1)
    ev = jnp.concatenate([ev, i < cnt_dst], axis=1)
    return cols, ev


def _knn(src_pos, cnt_src, dst_pos, k):
    d2 = ((dst_pos[:, None, :] - src_pos[None, :, :]) ** 2).sum(-1)
    j = jnp.arange(src_pos.shape[0])
    d2 = jnp.where(j[None, :] < cnt_src, d2, jnp.inf)
    _, idx = jax.lax.top_k(-d2, k)
    return idx


def _interp(x_src, pos_src, pos_dst, nn_idx):
    diff = pos_src[nn_idx] - pos_dst[:, None, :]
    d2 = (diff * diff).sum(-1, keepdims=True)
    w = 1.0 / jnp.maximum(d2, 1e-16)
    return (x_src[nn_idx] * w).sum(1) / w.sum(1)


def kernel(x, xgboost_score, pos, batch, params):
    score = xgboost_score
    over = score > _CUT
    mask_idx = jnp.nonzero(over, size=_N, fill_value=_N)[0].astype(jnp.int32)
    cnt0 = jnp.sum(over).astype(jnp.int32)
    p0 = pos[mask_idx]
    n1s = (_N + 1) // 2
    n2s = (2 * n1s + 4) // 5
    n3s = (2 * n2s + 4) // 5

    sel1 = _fps(p0, cnt0, n1s)
    n1 = jnp.maximum(1, (cnt0 + 1) // 2)
    p1 = p0[sel1]
    c1, v1 = _radius(p0, cnt0, p1, n1, 49.0)
    sel2 = _fps(p1, n1, n2s)
    n2 = jnp.maximum(1, (2 * n1 + 4) // 5)
    p2 = p1[sel2]
    c2, v2 = _radius(p1, n1, p2, n2, 256.0)
    sel3 = _fps(p2, n2, n3s)
    n3 = jnp.maximum(1, (2 * n2 + 4) // 5)
    p3 = p2[sel3]
    c3, v3 = _radius(p2, n2, p3, n3, 900.0)
    nn3 = jnp.zeros((n3s, 1), jnp.int32)
    nn2 = _knn(p3, n3, p2, 3)
    nn1 = _knn(p2, n2, p1, 3)
    nn0 = _knn(p1, n1, p0, 3)

    x0 = jnp.concatenate([x[mask_idx], score[mask_idx][:, None]], axis=1)
    x1 = _sa_conv(params['sa1'], x0, p0, p1, c1, v1)
    x2 = _sa_conv(params['sa2'], x1, p1, p2, c2, v2)
    x3 = _sa_conv(params['sa3'], x2, p2, p3, c3, v3)
    h4 = _mlp(jnp.concatenate([x3, p3], axis=1),
              [('bn', w) for w in params['sa4']])
    valid3 = jnp.arange(h4.shape[0]) < n3
    x4 = jnp.max(jnp.where(valid3[:, None], h4, -jnp.inf), axis=0,
                 keepdims=True)
    p4 = jnp.zeros((1, 3), x.dtype)
    y = _interp(x4, p4, p3, nn3)
    y = _mlp(jnp.concatenate([y, x3], axis=1),
             [('bn', w) for w in params['fp3']])
    y = _interp(y, p3, p2, nn2)
    y = _mlp(jnp.concatenate([y, x2], axis=1),
             [('bn', w) for w in params['fp2']])
    y = _interp(y, p2, p1, nn1)
    y = _mlp(jnp.concatenate([y, x1], axis=1),
             [('bn', w) for w in params['fp1']])
    y = _interp(y, p1, p0, nn0)
    head = ([('bn', w) for w in params['fp0']]
            + [('relu', params['lin1']), ('none', params['lin2']),
               ('none', params['lin3'])])
    h = _mlp(jnp.concatenate([y, x0], axis=1), head)
    out = jnp.zeros((pos.shape[0],), x.dtype).at[mask_idx].set(
        h[:, 0], mode='drop')
    return jax.nn.sigmoid(out)


# speedup vs baseline: 8.9800x; 1.1301x over previous
import functools

import jax
import jax.numpy as jnp
from jax.experimental import pallas as pl
from jax.experimental.pallas import tpu as pltpu

_CUT = 0.00394
_BN_EPS = 1e-5
_N = 10000


def _rup(x, m):
    return (x + m - 1) // m * m


def _fps_body(p_ref, cnt_ref, sel_ref, *, R, S, n_out):
    cnt = cnt_ref[0, 0]
    px = p_ref[0:R, :]
    py = p_ref[R:2 * R, :]
    pz = p_ref[2 * R:3 * R, :]
    idx = (jax.lax.broadcasted_iota(jnp.int32, (R, 128), 0) * 128
           + jax.lax.broadcasted_iota(jnp.int32, (R, 128), 1))
    sel_idx = (jax.lax.broadcasted_iota(jnp.int32, (S, 128), 0) * 128
               + jax.lax.broadcasted_iota(jnp.int32, (S, 128), 1))
    inf = jnp.float32(jnp.inf)
    dist0 = jnp.where(idx < cnt, inf, -inf)
    sel0 = jnp.zeros((S, 128), jnp.int32)

    def body(i, carry):
        dist, sel, cur = carry
        sel = jnp.where(sel_idx == i, cur, sel)
        curmask = idx == cur
        cx = jnp.sum(jnp.where(curmask, px, 0.0))
        cy = jnp.sum(jnp.where(curmask, py, 0.0))
        cz = jnp.sum(jnp.where(curmask, pz, 0.0))
        dx = px - cx
        dy = py - cy
        dz = pz - cz
        d = dx * dx + dy * dy + dz * dz
        dist = jnp.minimum(dist, d)
        m = jnp.max(dist)
        cur = jnp.min(jnp.where(dist == m, idx, jnp.int32(R * 128)))
        return dist, sel, cur

    _, sel, _ = jax.lax.fori_loop(0, n_out, body, (dist0, sel0, jnp.int32(0)),
                                  unroll=False)
    sel_ref[...] = sel


def _fps(p, cnt, n_out):
    n = p.shape[0]
    n_pad = _rup(n, 128)
    R = n_pad // 128
    S = _rup(pl.cdiv(n_out, 128), 8)
    pt = jnp.zeros((3, n_pad), jnp.float32).at[:, :n].set(p.T)
    pt = pt.reshape(3 * R, 128)
    cnt2 = jnp.asarray(cnt, jnp.int32).reshape(1, 1)
    out = pl.pallas_call(
        functools.partial(_fps_body, R=R, S=S, n_out=n_out),
        out_shape=jax.ShapeDtypeStruct((S, 128), jnp.int32),
        in_specs=[
            pl.BlockSpec(memory_space=pltpu.VMEM),
            pl.BlockSpec(memory_space=pltpu.SMEM),
        ],
        out_specs=pl.BlockSpec(memory_space=pltpu.VMEM),
    )(pt, cnt2)
    return out.reshape(-1)[:n_out]


def _conv_body(msg_ref, ev_ref, *args, T, K, nlayers):
    wr = args[:-1]
    out_ref = args[-1]
    h = msg_ref[...]
    for li in range(nlayers):
        w = wr[4 * li][...]
        b = wr[4 * li + 1][...]
        g = wr[4 * li + 2][...]
        be = wr[4 * li + 3][...]
        h = jnp.dot(h, w, preferred_element_type=jnp.float32) + b
        h = jnp.maximum(h, 0.0)
        h = (h / jnp.sqrt(jnp.float32(1.0 + _BN_EPS))) * g + be
    ev = ev_ref[...]
    h = jnp.where(ev > 0, h, -jnp.inf)
    for t in range(T):
        m = jnp.max(h[t * K:(t + 1) * K, :], axis=0, keepdims=True)
        out_ref[t:t + 1, :] = jnp.where(m == -jnp.inf, 0.0, m)


def _sa_conv(ps, x_src, pos_src, pos_dst, cols, ev, T=32, K=72):
    nd, k0 = cols.shape
    ndp = _rup(nd, T)
    cols_p = jnp.zeros((ndp, K), jnp.int32).at[:nd, :k0].set(cols)
    ev_p = jnp.zeros((ndp, K), jnp.bool_).at[:nd, :k0].set(ev)
    pos_dst_p = jnp.zeros((ndp, 3), pos_dst.dtype).at[:nd].set(pos_dst)
    col = cols_p.reshape(-1)
    row = jnp.repeat(jnp.arange(ndp, dtype=jnp.int32), K)
    msg = jnp.concatenate([x_src[col], pos_src[col] - pos_dst_p[row]], axis=1)
    evm = ev_p.reshape(-1, 1).astype(jnp.float32)
    cin = msg.shape[1]
    cout = ps[-1][0].shape[1]
    wr = []
    wspecs = []
    for (w, b, g, be) in ps:
        for a in (w, b.reshape(1, -1), g.reshape(1, -1), be.reshape(1, -1)):
            wr.append(a)
            wspecs.append(pl.BlockSpec(a.shape, lambda i: (0, 0)))
    grid = ndp // T
    out = pl.pallas_call(
        functools.partial(_conv_body, T=T, K=K, nlayers=len(ps)),
        grid=(grid,),
        out_shape=jax.ShapeDtypeStruct((ndp, cout), jnp.float32),
        in_specs=[
            pl.BlockSpec((T * K, cin), lambda i: (i, 0)),
            pl.BlockSpec((T * K, 1), lambda i: (i, 0)),
        ] + wspecs,
        out_specs=pl.BlockSpec((T, cout), lambda i: (i, 0)),
    )(msg, evm, *wr)
    return out[:nd]


def _mlp_body(x_ref, *args, specs):
    wr = args[:-1]
    out_ref = args[-1]
    h = x_ref[...]
    wi = 0
    for kind in specs:
        w = wr[wi][...]
        b = wr[wi + 1][...]
        wi += 2
        h = jnp.dot(h, w, preferred_element_type=jnp.float32) + b
        if kind == 'bn':
            h = jnp.maximum(h, 0.0)
            g = wr[wi][...]
            be = wr[wi + 1][...]
            wi += 2
            h = (h / jnp.sqrt(jnp.float32(1.0 + _BN_EPS))) * g + be
        elif kind == 'relu':
            h = jnp.maximum(h, 0.0)
    out_ref[...] = h


def _mlp(x, layers, tr=512):
    rn, cin = x.shape
    rp = _rup(rn, tr)
    xp = jnp.zeros((rp, cin), jnp.float32).at[:rn].set(x)
    wr = []
    wspecs = []
    specs = []
    for kind, ws in layers:
        specs.append(kind)
        tensors = [ws[0], ws[1].reshape(1, -1)]
        if kind == 'bn':
            tensors += [ws[2].reshape(1, -1), ws[3].reshape(1, -1)]
        for a in tensors:
            wr.append(a)
            wspecs.append(pl.BlockSpec(a.shape, lambda i: (0, 0)))
    cout = layers[-1][1][0].shape[1]
    grid = rp // tr
    out = pl.pallas_call(
        functools.partial(_mlp_body, specs=tuple(specs)),
        grid=(grid,),
        out_shape=jax.ShapeDtypeStruct((rp, cout), jnp.float32),
        in_specs=[pl.BlockSpec((tr, cin), lambda i: (i, 0))] + wspecs,
        out_specs=pl.BlockSpec((tr, cout), lambda i: (i, 0)),
    )(xp, *wr)
    return out[:rn]


def _hitcum_body(dst_ref, src_ref, cnt_ref, out_ref, *, r2):
    cnt = cnt_ref[0, 0]
    sx = src_ref[0:1, :]
    sy = src_ref[1:2, :]
    sz = src_ref[2:3, :]
    dx = dst_ref[:, 0:1] - sx
    dy = dst_ref[:, 1:2] - sy
    dz = dst_ref[:, 2:3] - sz
    d2 = dx * dx + dy * dy + dz * dz
    j = jax.lax.broadcasted_iota(jnp.int32, d2.shape, 1)
    hit = (d2 <= r2) & (j < cnt)
    c = hit.astype(jnp.int32)
    n = c.shape[1]
    s = 1
    while s < n:
        shifted = jnp.concatenate(
            [jnp.zeros((c.shape[0], s), jnp.int32), c[:, :-s]], axis=1)
        c = c + shifted
        s *= 2
    out_ref[...] = c.astype(jnp.int16)


def _radius(src_pos, cnt_src, dst_pos, cnt_dst, r2, max_nn=64):
    ns = src_pos.shape[0]
    nd = dst_pos.shape[0]
    tr = 32
    nsp = _rup(ns, 128)
    ndp = _rup(nd, tr)
    srcp = jnp.zeros((3, nsp), jnp.float32).at[:, :ns].set(src_pos.T)
    dstp = jnp.zeros((ndp, 3), jnp.float32).at[:nd].set(dst_pos)
    cnt2 = jnp.asarray(cnt_src, jnp.int32).reshape(1, 1)
    c = pl.pallas_call(
        functools.partial(_hitcum_body, r2=r2),
        grid=(ndp // tr,),
        out_shape=jax.ShapeDtypeStruct((ndp, nsp), jnp.int16),
        in_specs=[
            pl.BlockSpec((tr, 3), lambda i: (i, 0)),
            pl.BlockSpec((3, nsp), lambda i: (0, 0)),
            pl.BlockSpec(memory_space=pltpu.SMEM),
        ],
        out_specs=pl.BlockSpec((tr, nsp), lambda i: (i, 0)),
    )(dstp, srcp, cnt2)
    c = c[:nd, :ns]
    kq = jnp.broadcast_to(jnp.arange(1, max_nn + 1, dtype=jnp.int16)[None, :],
                          (nd, max_nn))
    cols = jax.vmap(jnp.searchsorted)(c, kq).astype(jnp.int32)
    i = jnp.arange(nd, dtype=jnp.int32)[:, None]
    ev = (cols < ns) & (cols != i) & (i < cnt_dst)
    cols = jnp.concatenate(
        [jnp.minimum(cols, ns - 1), jnp.broadcast_to(i, (nd, 1))], axis=1)
    ev = jnp.concatenate([ev, i < cnt_dst], axis=1)
    return cols, ev


def _knn(src_pos, cnt_src, dst_pos, k):
    ns = src_pos.shape[0]
    d2 = ((dst_pos[:, None, :] - src_pos[None, :, :]) ** 2).sum(-1)
    j = jnp.arange(ns)
    d2 = jnp.where(j[None, :] < cnt_src, d2, jnp.inf)
    bits = jax.lax.bitcast_convert_type(d2, jnp.int32)
    jr = jnp.arange(ns, dtype=jnp.int32)[None, :]
    picked = jnp.zeros(d2.shape, jnp.bool_)
    idxs = []
    for _ in range(k):
        bm = jnp.where(picked, jnp.int32(0x7FFFFFFF), bits)
        b = jnp.min(bm, axis=1, keepdims=True)
        m = jnp.min(jnp.where((bm == b) & ~picked, jr, jnp.int32(ns)),
                    axis=1, keepdims=True)
        idxs.append(m)
        picked = picked | (jr == m)
    return jnp.concatenate(idxs, axis=1)


def _interp(x_src, pos_src, pos_dst, nn_idx):
    diff = pos_src[nn_idx] - pos_dst[:, None, :]
    d2 = (diff * diff).sum(-1, keepdims=True)
    w = 1.0 / jnp.maximum(d2, 1e-16)
    return (x_src[nn_idx] * w).sum(1) / w.sum(1)


def kernel(x, xgboost_score, pos, batch, params):
    score = xgboost_score
    over = score > _CUT
    mask_idx = jnp.nonzero(over, size=_N, fill_value=_N)[0].astype(jnp.int32)
    cnt0 = jnp.sum(over).astype(jnp.int32)
    p0 = pos[mask_idx]
    n1s = (_N + 1) // 2
    n2s = (2 * n1s + 4) // 5
    n3s = (2 * n2s + 4) // 5

    sel1 = _fps(p0, cnt0, n1s)
    n1 = jnp.maximum(1, (cnt0 + 1) // 2)
    p1 = p0[sel1]
    c1, v1 = _radius(p0, cnt0, p1, n1, 49.0)
    sel2 = _fps(p1, n1, n2s)
    n2 = jnp.maximum(1, (2 * n1 + 4) // 5)
    p2 = p1[sel2]
    c2, v2 = _radius(p1, n1, p2, n2, 256.0)
    sel3 = _fps(p2, n2, n3s)
    n3 = jnp.maximum(1, (2 * n2 + 4) // 5)
    p3 = p2[sel3]
    c3, v3 = _radius(p2, n2, p3, n3, 900.0)
    nn3 = jnp.zeros((n3s, 1), jnp.int32)
    nn2 = _knn(p3, n3, p2, 3)
    nn1 = _knn(p2, n2, p1, 3)
    nn0 = _knn(p1, n1, p0, 3)

    x0 = jnp.concatenate([x[mask_idx], score[mask_idx][:, None]], axis=1)
    x1 = _sa_conv(params['sa1'], x0, p0, p1, c1, v1)
    x2 = _sa_conv(params['sa2'], x1, p1, p2, c2, v2)
    x3 = _sa_conv(params['sa3'], x2, p2, p3, c3, v3)
    h4 = _mlp(jnp.concatenate([x3, p3], axis=1),
              [('bn', w) for w in params['sa4']])
    valid3 = jnp.arange(h4.shape[0]) < n3
    x4 = jnp.max(jnp.where(valid3[:, None], h4, -jnp.inf), axis=0,
                 keepdims=True)
    p4 = jnp.zeros((1, 3), x.dtype)
    y = _interp(x4, p4, p3, nn3)
    y = _mlp(jnp.concatenate([y, x3], axis=1),
             [('bn', w) for w in params['fp3']])
    y = _interp(y, p3, p2, nn2)
    y = _mlp(jnp.concatenate([y, x2], axis=1),
             [('bn', w) for w in params['fp2']])
    y = _interp(y, p2, p1, nn1)
    y = _mlp(jnp.concatenate([y, x1], axis=1),
             [('bn', w) for w in params['fp1']])
    y = _interp(y, p1, p0, nn0)
    head = ([('bn', w) for w in params['fp0']]
            + [('relu', params['lin1']), ('none', params['lin2']),
               ('none', params['lin3'])])
    h = _mlp(jnp.concatenate([y, x0], axis=1), head)
    out = jnp.zeros((pos.shape[0],), x.dtype).at[mask_idx].set(
        h[:, 0], mode='drop')
    return jax.nn.sigmoid(out)
